# baseline (speedup 1.0000x reference)
import functools
import jax
import jax.numpy as jnp
from jax.experimental import pallas as pl
from jax.experimental.pallas import tpu as pltpu


def _roll(x, dist, axis):
    n = x.shape[axis]
    d = dist % n
    if axis == 0:
        return jnp.concatenate([x[n - d:, :], x[: n - d, :]], axis=0)
    return jnp.concatenate([x[:, n - d:], x[:, : n - d]], axis=1)


def _lovasz_kernel(pred_ref, tgt_ref, out_ref, *, logn, rb, lanes):
    c = pl.program_id(0)
    p = pred_ref[0]
    t = tgt_ref[...]
    rows = p.shape[0]

    fg = (t == c)
    e = jnp.where(fg, 1.0 - p, p)
    bits = jax.lax.bitcast_convert_type(e, jnp.int32)
    x = (bits << 1) | fg.astype(jnp.int32)

    row_i = jax.lax.broadcasted_iota(jnp.int32, (rows, lanes), 0)
    lane_i = jax.lax.broadcasted_iota(jnp.int32, (rows, lanes), 1)

    def idx_bit(b):
        if b < rb:
            return (row_i >> b) & 1
        return (lane_i >> (b - rb)) & 1

    pbit = [idx_bit(m) == 1 for m in range(logn)]
    flip = [idx_bit(k) - 1 for k in range(1, logn + 1)]

    x = x ^ flip[0]
    for k in range(1, logn + 1):
        if k > 1:
            x = x ^ (flip[k - 2] ^ flip[k - 1])
        for m in range(k - 1, -1, -1):
            if m < rb:
                j = 1 << m
                v = x.reshape(rows // (2 * j), 2, j, lanes)
                mn = jnp.minimum(v[:, 0], v[:, 1])
                mx = jnp.maximum(v[:, 0], v[:, 1])
                x = jnp.stack([mn, mx], axis=1).reshape(rows, lanes)
                continue
            if m < rb:
                axis, dist = 0, 1 << m
            else:
                axis, dist = 1, 1 << (m - rb)
            y = _roll(x, -dist, axis)
            mn = jnp.minimum(x, y)
            mx = jnp.maximum(x, y)
            x = jnp.where(pbit[m], _roll(mx, dist, axis), mn)
    x = x ^ flip[logn - 1]

    fs = (x & 1).astype(jnp.float32)
    es = jax.lax.bitcast_convert_type(
        jax.lax.shift_right_logical(x, 1), jnp.float32)

    g = jnp.sum(fs)
    csum = fs
    s = 1
    while s < rows:
        csum = csum + jnp.concatenate(
            [jnp.zeros((s, lanes), jnp.float32), csum[: rows - s]], axis=0)
        s *= 2
    col_tot = csum[rows - 1:rows, :]
    cp = jnp.concatenate([jnp.zeros((1, 1), jnp.float32),
                          col_tot[:, : lanes - 1]], axis=1)
    s = 1
    while s < lanes:
        cp = cp + jnp.concatenate(
            [jnp.zeros((1, s), jnp.float32), cp[:, : lanes - s]], axis=1)
        s *= 2
    k_arr = csum + cp

    n_arr = (lane_i * rows + row_i + 1).astype(jnp.float32)
    J = n_arr / (g + n_arr - k_arr)

    top_next = jnp.concatenate(
        [es[0:1, 1:], jnp.zeros((1, 1), jnp.float32)], axis=1)
    e_next = jnp.concatenate([es[1:], top_next], axis=0)

    loss_c = jnp.sum(J * (es - e_next))
    out_ref[...] = jnp.full(out_ref.shape, loss_c, jnp.float32)


@jax.jit
def kernel(pred, target):
    P, C = pred.shape
    lanes = 128
    rows = P // lanes
    logn = P.bit_length() - 1
    rb = rows.bit_length() - 1

    pred_t = pred.T.reshape(C, rows, lanes)
    tgt = target.astype(jnp.int32).reshape(rows, lanes)

    out = pl.pallas_call(
        functools.partial(_lovasz_kernel, logn=logn, rb=rb, lanes=lanes),
        grid=(C,),
        in_specs=[
            pl.BlockSpec((1, rows, lanes), lambda c: (c, 0, 0)),
            pl.BlockSpec((rows, lanes), lambda c: (0, 0)),
        ],
        out_specs=pl.BlockSpec((1, 8, 128), lambda c: (c, 0, 0)),
        out_shape=jax.ShapeDtypeStruct((C, 8, 128), jnp.float32),
        compiler_params=pltpu.CompilerParams(
            dimension_semantics=("parallel",)),
    )(pred_t, tgt)
    return jnp.mean(out[:, 0, 0])


# speedup vs baseline: 2.6728x; 2.6728x over previous
import functools
import jax
import jax.numpy as jnp
from jax.experimental import pallas as pl
from jax.experimental.pallas import tpu as pltpu


def _roll(x, dist, axis):
    n = x.shape[axis]
    d = dist % n
    if axis == 0:
        return jnp.concatenate([x[n - d:, :], x[: n - d, :]], axis=0)
    return jnp.concatenate([x[:, n - d:], x[:, : n - d]], axis=1)


def _lovasz_kernel(pred_ref, tgt_ref, out_ref, *, logn, rb, lanes):
    c = pl.program_id(0)
    p = pred_ref[0]
    t = tgt_ref[...]
    rows = p.shape[0]

    fg = (t == c)
    e = jnp.where(fg, 1.0 - p, p)
    bits = jax.lax.bitcast_convert_type(e, jnp.int32)
    x = (bits << 1) | fg.astype(jnp.int32)

    row_i = jax.lax.broadcasted_iota(jnp.int32, (rows, lanes), 0)
    lane_i = jax.lax.broadcasted_iota(jnp.int32, (rows, lanes), 1)

    t = 3 if rb > 6 else 0

    def perm(b):
        if b < t:
            return rb - t + b
        if rb - t <= b < rb:
            return b - (rb - t)
        return b

    def idx_bit(b):
        if b < rb:
            return (row_i >> perm(b)) & 1
        return (lane_i >> (b - rb)) & 1

    pbit = [idx_bit(m) == 1 for m in range(logn)]
    flip = [idx_bit(k) - 1 for k in range(1, logn + 1)]

    x = x ^ flip[0]
    for k in range(1, logn + 1):
        if k > 1:
            x = x ^ (flip[k - 2] ^ flip[k - 1])
        for m in range(k - 1, -1, -1):
            hb = perm(m) if m < rb else m
            if 3 <= hb < rb:
                j = 1 << hb
                v = x.reshape(rows // (2 * j), 2, j, lanes)
                mn = jnp.minimum(v[:, 0], v[:, 1])
                mx = jnp.maximum(v[:, 0], v[:, 1])
                x = jnp.stack([mn, mx], axis=1).reshape(rows, lanes)
                continue
            if hb < rb:
                axis, dist = 0, 1 << hb
            else:
                axis, dist = 1, 1 << (hb - rb)
            y = _roll(x, -dist, axis)
            mn = jnp.minimum(x, y)
            mx = jnp.maximum(x, y)
            x = jnp.where(pbit[m], _roll(mx, dist, axis), mn)
    x = x ^ flip[logn - 1]

    for b in range(t):
        hi = rb - t + b
        d = (1 << hi) - (1 << b)
        sel_lo = ((row_i >> b) & 1) != ((row_i >> hi) & 1)
        up = (row_i >> hi) & 1
        y1 = _roll(x, -d, 0)
        y2 = _roll(x, d, 0)
        x = jnp.where(sel_lo, jnp.where(up == 1, y2, y1), x)

    fs = (x & 1).astype(jnp.float32)
    es = jax.lax.bitcast_convert_type(
        jax.lax.shift_right_logical(x, 1), jnp.float32)

    g = jnp.sum(fs)
    csum = fs
    s = 1
    while s < rows:
        csum = csum + jnp.concatenate(
            [jnp.zeros((s, lanes), jnp.float32), csum[: rows - s]], axis=0)
        s *= 2
    col_tot = csum[rows - 1:rows, :]
    cp = jnp.concatenate([jnp.zeros((1, 1), jnp.float32),
                          col_tot[:, : lanes - 1]], axis=1)
    s = 1
    while s < lanes:
        cp = cp + jnp.concatenate(
            [jnp.zeros((1, s), jnp.float32), cp[:, : lanes - s]], axis=1)
        s *= 2
    k_arr = csum + cp

    n_arr = (lane_i * rows + row_i + 1).astype(jnp.float32)
    J = n_arr / (g + n_arr - k_arr)

    top_next = jnp.concatenate(
        [es[0:1, 1:], jnp.zeros((1, 1), jnp.float32)], axis=1)
    e_next = jnp.concatenate([es[1:], top_next], axis=0)

    loss_c = jnp.sum(J * (es - e_next))
    out_ref[...] = jnp.full(out_ref.shape, loss_c, jnp.float32)


@jax.jit
def kernel(pred, target):
    P, C = pred.shape
    lanes = 128
    rows = P // lanes
    logn = P.bit_length() - 1
    rb = rows.bit_length() - 1

    pred_t = pred.T.reshape(C, rows, lanes)
    tgt = target.astype(jnp.int32).reshape(rows, lanes)

    out = pl.pallas_call(
        functools.partial(_lovasz_kernel, logn=logn, rb=rb, lanes=lanes),
        grid=(C,),
        in_specs=[
            pl.BlockSpec((1, rows, lanes), lambda c: (c, 0, 0)),
            pl.BlockSpec((rows, lanes), lambda c: (0, 0)),
        ],
        out_specs=pl.BlockSpec((1, 8, 128), lambda c: (c, 0, 0)),
        out_shape=jax.ShapeDtypeStruct((C, 8, 128), jnp.float32),
        compiler_params=pltpu.CompilerParams(
            dimension_semantics=("parallel",)),
    )(pred_t, tgt)
    return jnp.mean(out[:, 0, 0])
